# baseline (speedup 1.0000x reference)
import jax
import jax.numpy as jnp
from jax import lax
from jax.experimental import pallas as pl
from jax.experimental.pallas import tpu as pltpu
from jax.experimental.pallas import tpu_sc as plsc

N = 10000
E = 160000
D = 128
NC = 2
NS = 16
NW = NC * NS
SLAB_R = 40
SLAB_W = 128
CE = 32
CPG = SLAB_W // CE
NGRP = SLAB_R
EW = SLAB_R * SLAB_W
EPAD = EW * NW
PAD_DST = N
ACC = 10240
RPT = ACC // NS
CW = 16

_mesh = plsc.VectorSubcoreMesh(
    core_axis_name="c", subcore_axis_name="s", num_cores=NC, num_subcores=NS)


ZC = 128


def _zero_rows(rows):
    zeros16 = jnp.zeros((16,), jnp.float32)

    def _z(i, _):
        rows[i // (D // 16), pl.ds((i % (D // 16)) * 16, 16)] = zeros16
        return 0
    lax.fori_loop(0, ZC * (D // 16), _z, 0)


def _conv_body(x_hbm, src_hbm, dst_hbm, out_hbm,
               idx_s, idx_d, gA0, gA1, gA2, gA3, gB0, gB1, gB2, gB3,
               idx_c, rowsA, rowsB, acc, semA, semB):
    gA = (gA0, gA1, gA2, gA3)
    gB = (gB0, gB1, gB2, gB3)
    cx = lax.axis_index("c")
    s = lax.axis_index("s")
    w = s * NC + cx

    pltpu.sync_copy(src_hbm.at[w], idx_s)
    pltpu.sync_copy(dst_hbm.at[w], idx_d)

    def _stage32(slab, q, b, buf):
        for k in range(CE // 16):
            buf[pl.ds(k * 16, 16)] = slab[q, pl.ds(b * CE + k * 16, 16)]

    def _issue(q, g, rows, sem):
        for b in range(CPG):
            _stage32(idx_s, q, b, g[b])
            pltpu.async_copy(x_hbm.at[g[b]], rows.at[pl.ds(b * CE, CE)], sem)

    def _drain_scatter(q, g, rows, sem):
        for b in range(CPG):
            pltpu.make_async_copy(x_hbm.at[g[b]],
                                  rows.at[pl.ds(b * CE, CE)], sem).wait()
        for b in range(CPG):
            _stage32(idx_d, q, b, idx_c)

    _issue(0, gA, rowsA, semA)

    _zero_rows(rowsB)

    def _zacc(i, _):
        pltpu.sync_copy(rowsB.at[pl.ds(0, ZC)],
                        acc.at[pl.ds(s * RPT + i * ZC, ZC)])
        return 0
    lax.fori_loop(0, RPT // ZC, _zacc, 0)

    plsc.subcore_barrier()

    def _outer(p, _):
        qa = 2 * p
        _issue(qa + 1, gB, rowsB, semB)
        _drain_scatter(qa, gA, rowsA, semA)

        @pl.when(qa + 2 < NGRP)
        def _():
            _issue(qa + 2, gA, rowsA, semA)

        _drain_scatter(qa + 1, gB, rowsB, semB)
        return 0
    lax.fori_loop(0, NGRP // 2, _outer, 0)

    plsc.subcore_barrier()

    def _rd(i, _):
        pltpu.sync_copy(acc.at[pl.ds(s * RPT + i * ZC, ZC)],
                        rowsA.at[pl.ds(0, ZC)])
        pltpu.sync_copy(rowsA.at[pl.ds(0, ZC)],
                        out_hbm.at[cx, pl.ds(s * RPT + i * ZC, ZC)])
        return 0
    lax.fori_loop(0, RPT // ZC, _rd, 0)


_conv = pl.kernel(
    _conv_body,
    out_type=[
        jax.ShapeDtypeStruct((NC, ACC, D), jnp.float32),
    ],
    mesh=_mesh,
    scratch_types=(
        [pltpu.VMEM((SLAB_R, SLAB_W), jnp.int32)] * 2
        + [pltpu.VMEM((CE,), jnp.int32)] * 9
        + [pltpu.VMEM((CPG * CE, D), jnp.float32)] * 2
        + [pltpu.VMEM_SHARED((ACC, D), jnp.float32),
           pltpu.SemaphoreType.DMA,
           pltpu.SemaphoreType.DMA]
    ),
)


def _pad_edges(ei):
    pad = EPAD - E
    src = jnp.concatenate([ei[0], jnp.zeros((pad,), jnp.int32)])
    dst = jnp.concatenate([ei[1], jnp.full((pad,), PAD_DST, jnp.int32)])
    return src.reshape(NW, SLAB_R, SLAB_W), dst.reshape(NW, SLAB_R, SLAB_W)


_EB = 1000


def _cnt_tc_body(da, db, oa, ob):
    i = pl.program_id(0)

    @pl.when(i == 0)
    def _init():
        oa[...] = jnp.zeros_like(oa)
        ob[...] = jnp.zeros_like(ob)

    lo_iota = lax.broadcasted_iota(jnp.int32, (_EB, D), 1)

    def hist(d):
        lo = (d % D == lo_iota).astype(jnp.float32)
        hi = (d // D == lo_iota).astype(jnp.float32)
        return lax.dot_general(hi, lo, (((0,), (0,)), ((), ())),
                               preferred_element_type=jnp.float32)

    oa[...] += hist(da[...])
    ob[...] += hist(db[...])


def _cnt_tc(dsta, dstb):
    return pl.pallas_call(
        _cnt_tc_body,
        grid=(E // _EB,),
        in_specs=[
            pl.BlockSpec((_EB, 1), lambda i: (i, 0)),
            pl.BlockSpec((_EB, 1), lambda i: (i, 0)),
        ],
        out_specs=[
            pl.BlockSpec((D, D), lambda i: (0, 0)),
            pl.BlockSpec((D, D), lambda i: (0, 0)),
        ],
        out_shape=[
            jax.ShapeDtypeStruct((D, D), jnp.float32),
            jax.ShapeDtypeStruct((D, D), jnp.float32),
        ],
    )(dsta.reshape(E, 1), dstb.reshape(E, 1))


_RB = 400


def _lin_body(a0, a1, cn, x, wl, wr, b, o):
    cnt = jnp.maximum(cn[...], 1.0)
    mean = (a0[0] + a1[0]) / cnt
    h = (jnp.dot(mean, wl[...], preferred_element_type=jnp.float32)
         + jnp.dot(x[...], wr[...], preferred_element_type=jnp.float32)
         + b[...])
    o[...] = jnp.maximum(h, 0.0)


def _sage_linear(agg, cnt, x, wl, wr, b):
    return pl.pallas_call(
        _lin_body,
        grid=(N // _RB,),
        in_specs=[
            pl.BlockSpec((1, _RB, D), lambda i: (0, i, 0)),
            pl.BlockSpec((1, _RB, D), lambda i: (1, i, 0)),
            pl.BlockSpec((_RB, 1), lambda i: (i, 0)),
            pl.BlockSpec((_RB, D), lambda i: (i, 0)),
            pl.BlockSpec((D, D), lambda i: (0, 0)),
            pl.BlockSpec((D, D), lambda i: (0, 0)),
            pl.BlockSpec((1, D), lambda i: (0, 0)),
        ],
        out_specs=pl.BlockSpec((_RB, D), lambda i: (i, 0)),
        out_shape=jax.ShapeDtypeStruct((N, D), jnp.float32),
    )(agg, agg, cnt, x, wl, wr, b.reshape(1, D))


def _final_body(a, cn, ht, wl, wr, b, gnw, gnb, gms, pw1, pb1, pw2, pb2, o):
    cnt = jnp.maximum(cn[...], 1.0)
    mean = (a[0] + a[1]) / cnt
    x = (jnp.dot(mean, wl[...], preferred_element_type=jnp.float32)
         + jnp.dot(ht[...], wr[...], preferred_element_type=jnp.float32)
         + b[...])
    mu = jnp.mean(x, axis=0, keepdims=True)
    cen = x - mu * gms[...]
    var = jnp.mean(cen * cen, axis=0, keepdims=True)
    x = gnw[...] * cen / jnp.sqrt(var + 1e-5) + gnb[...]
    x = jnp.maximum(jnp.dot(x, pw1[...], preferred_element_type=jnp.float32)
                    + pb1[...], 0.0)
    x = jnp.dot(x, pw2[...], preferred_element_type=jnp.float32) + pb2[...]
    nrm = jnp.sqrt(jnp.sum(x * x, axis=1, keepdims=True))
    o[...] = x / jnp.maximum(nrm, 1e-12)


def _final_stage(agg2, cntb, h_tab, wl, wr, b,
                 gnw, gnb, gms, pw1, pb1, pw2, pb2):
    r1 = lambda v: v.reshape(1, D)
    return pl.pallas_call(
        _final_body,
        grid=(1,),
        in_specs=[
            pl.BlockSpec((NC, N, D), lambda i: (0, 0, 0)),
            pl.BlockSpec((N, 1), lambda i: (0, 0)),
            pl.BlockSpec((N, D), lambda i: (0, 0)),
            pl.BlockSpec((D, D), lambda i: (0, 0)),
            pl.BlockSpec((D, D), lambda i: (0, 0)),
            pl.BlockSpec((1, D), lambda i: (0, 0)),
            pl.BlockSpec((1, D), lambda i: (0, 0)),
            pl.BlockSpec((1, D), lambda i: (0, 0)),
            pl.BlockSpec((1, D), lambda i: (0, 0)),
            pl.BlockSpec((D, D), lambda i: (0, 0)),
            pl.BlockSpec((1, D), lambda i: (0, 0)),
            pl.BlockSpec((D, D), lambda i: (0, 0)),
            pl.BlockSpec((1, D), lambda i: (0, 0)),
        ],
        out_specs=pl.BlockSpec((N, D), lambda i: (0, 0)),
        out_shape=jax.ShapeDtypeStruct((N, D), jnp.float32),
    )(agg2, cntb, h_tab, wl, wr, r1(b), r1(gnw), r1(gnb), r1(gms),
      pw1, r1(pb1), pw2, r1(pb2))


def kernel(x_table, x_column, edge_index_t2c, edge_index_c2t,
           Wl1_t2c, Wr1_t2c, b1_t2c, Wl1_c2t, Wr1_c2t, b1_c2t,
           Wl2_t2c, Wr2_t2c, b2_t2c, Wl2_c2t, Wr2_c2t, b2_c2t,
           gn_weight, gn_bias, gn_mean_scale, pW1, pb1, pW2, pb2):
    srcA, dstA = _pad_edges(edge_index_t2c)
    srcB, dstB = _pad_edges(edge_index_c2t)

    cntA_m, cntB_m = _cnt_tc(edge_index_t2c[1], edge_index_c2t[1])
    cntA = cntA_m.reshape(D * D, 1)
    cntB = cntB_m.reshape(D * D, 1)
    (aggA,) = _conv(x_table, srcA, dstA)
    (aggB,) = _conv(x_column, srcB, dstB)

    h_col = _sage_linear(aggA, cntA, x_column, Wl1_t2c, Wr1_t2c, b1_t2c)
    h_tab = _sage_linear(aggB, cntB, x_table, Wl1_c2t, Wr1_c2t, b1_c2t)

    (agg2,) = _conv(h_col, srcB, dstB)

    return _final_stage(agg2, cntB, h_tab, Wl2_c2t, Wr2_c2t, b2_c2t,
                        gn_weight, gn_bias, gn_mean_scale,
                        pW1, pb1, pW2, pb2)


# speedup vs baseline: 1.0500x; 1.0500x over previous
import jax
import jax.numpy as jnp
from jax import lax
from jax.experimental import pallas as pl
from jax.experimental.pallas import tpu as pltpu
from jax.experimental.pallas import tpu_sc as plsc

N = 10000
E = 160000
D = 128
NC = 2
NS = 16
NW = NC * NS
SLAB_W = 128
ROWS_TOT = 1280
R_FAST = 64
R_SLOW = 16
CE = 32
GPR = 2
EPAD = ROWS_TOT * SLAB_W
PAD_DST = N
ACC = 10240
RPT = ACC // NS
CW = 16

_mesh = plsc.VectorSubcoreMesh(
    core_axis_name="c", subcore_axis_name="s", num_cores=NC, num_subcores=NS)


ZC = 64


def _zero_rows(rows):
    zeros16 = jnp.zeros((16,), jnp.float32)

    def _z(i, _):
        rows[i // (D // 16), pl.ds((i % (D // 16)) * 16, 16)] = zeros16
        return 0
    lax.fori_loop(0, ZC * (D // 16), _z, 0)


def _conv_body(x_hbm, src_hbm, dst_hbm, out_hbm,
               idx_s, idx_d, gA0, gA1, gB0, gB1,
               idx_c, rowsA, rowsB, acc, semA, semB):
    gA = (gA0, gA1)
    gB = (gB0, gB1)
    cx = lax.axis_index("c")
    s = lax.axis_index("s")

    _zero_rows(rowsB)

    def _zacc(i, _):
        pltpu.sync_copy(rowsB.at[pl.ds(0, ZC)],
                        acc.at[pl.ds(s * RPT + i * ZC, ZC)])
        return 0
    lax.fori_loop(0, RPT // ZC, _zacc, 0)

    plsc.subcore_barrier()

    def _stage32(slab, q, b, buf):
        row = q // GPR
        col = (q % GPR) * 2 + b
        for k in range(CE // 16):
            buf[pl.ds(k * 16, 16)] = slab[row, pl.ds(col * CE + k * 16, 16)]

    def _issue(q, g, rows, sem):
        for b in range(2):
            _stage32(idx_s, q, b, g[b])
            pltpu.async_copy(x_hbm.at[g[b]], rows.at[pl.ds(b * CE, CE)], sem)

    def _drain_scatter(q, g, rows, sem):
        for b in range(2):
            pltpu.make_async_copy(x_hbm.at[g[b]],
                                  rows.at[pl.ds(b * CE, CE)], sem).wait()
        for b in range(2):
            _stage32(idx_d, q, b, idx_c)
            pltpu.sync_copy(rows.at[pl.ds(b * CE, CE)], acc.at[idx_c],
                            add=True)

    def _run(nrows, base):
        pltpu.sync_copy(src_hbm.at[pl.ds(base, nrows)],
                        idx_s.at[pl.ds(0, nrows)])
        pltpu.sync_copy(dst_hbm.at[pl.ds(base, nrows)],
                        idx_d.at[pl.ds(0, nrows)])
        ngrp = nrows * GPR
        _issue(0, gA, rowsA, semA)

        def _outer(p, _):
            qa = 2 * p
            _issue(qa + 1, gB, rowsB, semB)
            _drain_scatter(qa, gA, rowsA, semA)

            @pl.when(qa + 2 < ngrp)
            def _():
                _issue(qa + 2, gA, rowsA, semA)

            _drain_scatter(qa + 1, gB, rowsB, semB)
            return 0
        lax.fori_loop(0, ngrp // 2, _outer, 0)

    @pl.when(cx == 0)
    def _fast():
        _run(R_FAST, s * R_FAST)

    @pl.when(cx == 1)
    def _slow():
        _run(R_SLOW, NS * R_FAST + s * R_SLOW)

    plsc.subcore_barrier()

    def _rd(i, _):
        pltpu.sync_copy(acc.at[pl.ds(s * RPT + i * ZC, ZC)],
                        rowsA.at[pl.ds(0, ZC)])
        pltpu.sync_copy(rowsA.at[pl.ds(0, ZC)],
                        out_hbm.at[cx, pl.ds(s * RPT + i * ZC, ZC)])
        return 0
    lax.fori_loop(0, RPT // ZC, _rd, 0)


_conv = pl.kernel(
    _conv_body,
    out_type=[
        jax.ShapeDtypeStruct((NC, ACC, D), jnp.float32),
    ],
    mesh=_mesh,
    scratch_types=(
        [pltpu.VMEM((R_FAST, SLAB_W), jnp.int32)] * 2
        + [pltpu.VMEM((CE,), jnp.int32)] * 5
        + [pltpu.VMEM((ZC, D), jnp.float32)] * 2
        + [pltpu.VMEM_SHARED((ACC, D), jnp.float32),
           pltpu.SemaphoreType.DMA,
           pltpu.SemaphoreType.DMA]
    ),
)


def _pad_edges(ei):
    pad = EPAD - E
    src = jnp.concatenate([ei[0], jnp.zeros((pad,), jnp.int32)])
    dump = PAD_DST + (jnp.arange(pad, dtype=jnp.int32) % (ACC - N - 16))
    dst = jnp.concatenate([ei[1], dump])
    return src.reshape(ROWS_TOT, SLAB_W), dst.reshape(ROWS_TOT, SLAB_W)


_EB = 1000


def _cnt_tc_body(da, db, oa, ob):
    i = pl.program_id(0)

    @pl.when(i == 0)
    def _init():
        oa[...] = jnp.zeros_like(oa)
        ob[...] = jnp.zeros_like(ob)

    lo_iota = lax.broadcasted_iota(jnp.int32, (_EB, D), 1)

    def hist(d):
        lo = (d % D == lo_iota).astype(jnp.float32)
        hi = (d // D == lo_iota).astype(jnp.float32)
        return lax.dot_general(hi, lo, (((0,), (0,)), ((), ())),
                               preferred_element_type=jnp.float32)

    oa[...] += hist(da[...])
    ob[...] += hist(db[...])


def _cnt_tc(dsta, dstb):
    return pl.pallas_call(
        _cnt_tc_body,
        grid=(E // _EB,),
        in_specs=[
            pl.BlockSpec((_EB, 1), lambda i: (i, 0)),
            pl.BlockSpec((_EB, 1), lambda i: (i, 0)),
        ],
        out_specs=[
            pl.BlockSpec((D, D), lambda i: (0, 0)),
            pl.BlockSpec((D, D), lambda i: (0, 0)),
        ],
        out_shape=[
            jax.ShapeDtypeStruct((D, D), jnp.float32),
            jax.ShapeDtypeStruct((D, D), jnp.float32),
        ],
    )(dsta.reshape(E, 1), dstb.reshape(E, 1))


_RB = 400


def _lin_body(a0, a1, cn, x, wl, wr, b, o):
    cnt = jnp.maximum(cn[...], 1.0)
    mean = (a0[0] + a1[0]) / cnt
    h = (jnp.dot(mean, wl[...], preferred_element_type=jnp.float32)
         + jnp.dot(x[...], wr[...], preferred_element_type=jnp.float32)
         + b[...])
    o[...] = jnp.maximum(h, 0.0)


def _sage_linear(agg, cnt, x, wl, wr, b):
    return pl.pallas_call(
        _lin_body,
        grid=(N // _RB,),
        in_specs=[
            pl.BlockSpec((1, _RB, D), lambda i: (0, i, 0)),
            pl.BlockSpec((1, _RB, D), lambda i: (1, i, 0)),
            pl.BlockSpec((_RB, 1), lambda i: (i, 0)),
            pl.BlockSpec((_RB, D), lambda i: (i, 0)),
            pl.BlockSpec((D, D), lambda i: (0, 0)),
            pl.BlockSpec((D, D), lambda i: (0, 0)),
            pl.BlockSpec((1, D), lambda i: (0, 0)),
        ],
        out_specs=pl.BlockSpec((_RB, D), lambda i: (i, 0)),
        out_shape=jax.ShapeDtypeStruct((N, D), jnp.float32),
    )(agg, agg, cnt, x, wl, wr, b.reshape(1, D))


def _final_body(a, cn, ht, wl, wr, b, gnw, gnb, gms, pw1, pb1, pw2, pb2, o):
    cnt = jnp.maximum(cn[...], 1.0)
    mean = (a[0] + a[1]) / cnt
    x = (jnp.dot(mean, wl[...], preferred_element_type=jnp.float32)
         + jnp.dot(ht[...], wr[...], preferred_element_type=jnp.float32)
         + b[...])
    mu = jnp.mean(x, axis=0, keepdims=True)
    cen = x - mu * gms[...]
    var = jnp.mean(cen * cen, axis=0, keepdims=True)
    x = gnw[...] * cen / jnp.sqrt(var + 1e-5) + gnb[...]
    x = jnp.maximum(jnp.dot(x, pw1[...], preferred_element_type=jnp.float32)
                    + pb1[...], 0.0)
    x = jnp.dot(x, pw2[...], preferred_element_type=jnp.float32) + pb2[...]
    nrm = jnp.sqrt(jnp.sum(x * x, axis=1, keepdims=True))
    o[...] = x / jnp.maximum(nrm, 1e-12)


def _final_stage(agg2, cntb, h_tab, wl, wr, b,
                 gnw, gnb, gms, pw1, pb1, pw2, pb2):
    r1 = lambda v: v.reshape(1, D)
    return pl.pallas_call(
        _final_body,
        grid=(1,),
        in_specs=[
            pl.BlockSpec((NC, N, D), lambda i: (0, 0, 0)),
            pl.BlockSpec((N, 1), lambda i: (0, 0)),
            pl.BlockSpec((N, D), lambda i: (0, 0)),
            pl.BlockSpec((D, D), lambda i: (0, 0)),
            pl.BlockSpec((D, D), lambda i: (0, 0)),
            pl.BlockSpec((1, D), lambda i: (0, 0)),
            pl.BlockSpec((1, D), lambda i: (0, 0)),
            pl.BlockSpec((1, D), lambda i: (0, 0)),
            pl.BlockSpec((1, D), lambda i: (0, 0)),
            pl.BlockSpec((D, D), lambda i: (0, 0)),
            pl.BlockSpec((1, D), lambda i: (0, 0)),
            pl.BlockSpec((D, D), lambda i: (0, 0)),
            pl.BlockSpec((1, D), lambda i: (0, 0)),
        ],
        out_specs=pl.BlockSpec((N, D), lambda i: (0, 0)),
        out_shape=jax.ShapeDtypeStruct((N, D), jnp.float32),
    )(agg2, cntb, h_tab, wl, wr, r1(b), r1(gnw), r1(gnb), r1(gms),
      pw1, r1(pb1), pw2, r1(pb2))


def kernel(x_table, x_column, edge_index_t2c, edge_index_c2t,
           Wl1_t2c, Wr1_t2c, b1_t2c, Wl1_c2t, Wr1_c2t, b1_c2t,
           Wl2_t2c, Wr2_t2c, b2_t2c, Wl2_c2t, Wr2_c2t, b2_c2t,
           gn_weight, gn_bias, gn_mean_scale, pW1, pb1, pW2, pb2):
    srcA, dstA = _pad_edges(edge_index_t2c)
    srcB, dstB = _pad_edges(edge_index_c2t)

    cntA_m, cntB_m = _cnt_tc(edge_index_t2c[1], edge_index_c2t[1])
    cntA = cntA_m.reshape(D * D, 1)
    cntB = cntB_m.reshape(D * D, 1)
    (aggA,) = _conv(x_table, srcA, dstA)
    (aggB,) = _conv(x_column, srcB, dstB)

    h_col = _sage_linear(aggA, cntA, x_column, Wl1_t2c, Wr1_t2c, b1_t2c)
    h_tab = _sage_linear(aggB, cntB, x_table, Wl1_c2t, Wr1_c2t, b1_c2t)

    (agg2,) = _conv(h_col, srcB, dstB)

    return _final_stage(agg2, cntB, h_tab, Wl2_c2t, Wr2_c2t, b2_c2t,
                        gn_weight, gn_bias, gn_mean_scale,
                        pW1, pb1, pW2, pb2)


# speedup vs baseline: 1.0890x; 1.0372x over previous
import jax
import jax.numpy as jnp
from jax import lax
from jax.experimental import pallas as pl
from jax.experimental.pallas import tpu as pltpu
from jax.experimental.pallas import tpu_sc as plsc

N = 10000
E = 160000
D = 128
NC = 2
NS = 16
NW = NC * NS
SLAB_W = 128
ROWS_TOT = 1280
R_FAST = 64
R_SLOW = 16
CE = 32
GPR = 2
EPAD = ROWS_TOT * SLAB_W
PAD_DST = N
ACC = 10240
RPT = ACC // NS
CW = 16

_mesh = plsc.VectorSubcoreMesh(
    core_axis_name="c", subcore_axis_name="s", num_cores=NC, num_subcores=NS)


ZC = 64


def _zero_rows(rows):
    zeros16 = jnp.zeros((16,), jnp.float32)

    def _z(i, _):
        rows[i // (D // 16), pl.ds((i % (D // 16)) * 16, 16)] = zeros16
        return 0
    lax.fori_loop(0, ZC * (D // 16), _z, 0)


def _conv_body(x_hbm, src_hbm, dst_hbm, out_hbm,
               idx_s, idx_d, gA0, gA1, gB0, gB1,
               idx_c, rowsA, rowsB, acc, semA, semB):
    gA = (gA0, gA1)
    gB = (gB0, gB1)
    cx = lax.axis_index("c")
    s = lax.axis_index("s")

    _zero_rows(rowsB)

    def _zacc(i, _):
        pltpu.sync_copy(rowsB.at[pl.ds(0, ZC)],
                        acc.at[pl.ds(s * RPT + i * ZC, ZC)])
        return 0
    lax.fori_loop(0, RPT // ZC, _zacc, 0)

    plsc.subcore_barrier()

    def _stage32(slab, q, b, buf):
        row = q // GPR
        col = (q % GPR) * 2 + b
        for k in range(CE // 16):
            buf[pl.ds(k * 16, 16)] = slab[row, pl.ds(col * CE + k * 16, 16)]

    def _issue(q, g, rows, sem):
        for b in range(2):
            _stage32(idx_s, q, b, g[b])
            pltpu.async_copy(x_hbm.at[g[b]], rows.at[pl.ds(b * CE, CE)], sem)

    def _drain_scatter(q, g, rows, sem):
        for b in range(2):
            pltpu.make_async_copy(x_hbm.at[g[b]],
                                  rows.at[pl.ds(b * CE, CE)], sem).wait()
        for b in range(2):
            _stage32(idx_d, q, b, idx_c)
            pltpu.sync_copy(rows.at[pl.ds(b * CE, CE)], acc.at[idx_c],
                            add=True)

    def _run(nrows, base):
        pltpu.sync_copy(src_hbm.at[pl.ds(base, nrows)],
                        idx_s.at[pl.ds(0, nrows)])
        pltpu.sync_copy(dst_hbm.at[pl.ds(base, nrows)],
                        idx_d.at[pl.ds(0, nrows)])
        ngrp = nrows * GPR
        _issue(0, gA, rowsA, semA)

        def _outer(p, _):
            qa = 2 * p
            _issue(qa + 1, gB, rowsB, semB)
            _drain_scatter(qa, gA, rowsA, semA)

            @pl.when(qa + 2 < ngrp)
            def _():
                _issue(qa + 2, gA, rowsA, semA)

            _drain_scatter(qa + 1, gB, rowsB, semB)
            return 0
        lax.fori_loop(0, ngrp // 2, _outer, 0)

    @pl.when(cx == 0)
    def _fast():
        _run(R_FAST, s * R_FAST)

    @pl.when(cx == 1)
    def _slow():
        _run(R_SLOW, NS * R_FAST + s * R_SLOW)

    plsc.subcore_barrier()

    nrd = RPT // ZC
    for i in range(nrd):
        buf = rowsA if i % 2 == 0 else rowsB
        sem = semA if i % 2 == 0 else semB
        dst = out_hbm.at[cx, pl.ds(s * RPT + i * ZC, ZC)]
        if i >= 2:
            prev = out_hbm.at[cx, pl.ds(s * RPT + (i - 2) * ZC, ZC)]
            pltpu.make_async_copy(buf.at[pl.ds(0, ZC)], prev, sem).wait()
        pltpu.sync_copy(acc.at[pl.ds(s * RPT + i * ZC, ZC)],
                        buf.at[pl.ds(0, ZC)])
        pltpu.async_copy(buf.at[pl.ds(0, ZC)], dst, sem)
    for i in (nrd - 2, nrd - 1):
        buf = rowsA if i % 2 == 0 else rowsB
        sem = semA if i % 2 == 0 else semB
        dst = out_hbm.at[cx, pl.ds(s * RPT + i * ZC, ZC)]
        pltpu.make_async_copy(buf.at[pl.ds(0, ZC)], dst, sem).wait()


_conv = pl.kernel(
    _conv_body,
    out_type=[
        jax.ShapeDtypeStruct((NC, ACC, D), jnp.float32),
    ],
    mesh=_mesh,
    scratch_types=(
        [pltpu.VMEM((R_FAST, SLAB_W), jnp.int32)] * 2
        + [pltpu.VMEM((CE,), jnp.int32)] * 5
        + [pltpu.VMEM((ZC, D), jnp.float32)] * 2
        + [pltpu.VMEM_SHARED((ACC, D), jnp.float32),
           pltpu.SemaphoreType.DMA,
           pltpu.SemaphoreType.DMA]
    ),
)


def _pad_edges(ei):
    pad = EPAD - E
    src = jnp.concatenate([ei[0], jnp.zeros((pad,), jnp.int32)])
    dump = PAD_DST + (jnp.arange(pad, dtype=jnp.int32) % (ACC - N - 16))
    dst = jnp.concatenate([ei[1], dump])
    return src.reshape(ROWS_TOT, SLAB_W), dst.reshape(ROWS_TOT, SLAB_W)


_EB = 1000


def _cnt_tc_body(da, db, oa, ob):
    i = pl.program_id(0)

    @pl.when(i == 0)
    def _init():
        oa[...] = jnp.zeros_like(oa)
        ob[...] = jnp.zeros_like(ob)

    lo_iota = lax.broadcasted_iota(jnp.int32, (_EB, D), 1)

    def hist(d):
        lo = (d % D == lo_iota).astype(jnp.float32)
        hi = (d // D == lo_iota).astype(jnp.float32)
        return lax.dot_general(hi, lo, (((0,), (0,)), ((), ())),
                               preferred_element_type=jnp.float32)

    oa[...] += hist(da[...])
    ob[...] += hist(db[...])


def _cnt_tc(dsta, dstb):
    return pl.pallas_call(
        _cnt_tc_body,
        grid=(E // _EB,),
        in_specs=[
            pl.BlockSpec((_EB, 1), lambda i: (i, 0)),
            pl.BlockSpec((_EB, 1), lambda i: (i, 0)),
        ],
        out_specs=[
            pl.BlockSpec((D, D), lambda i: (0, 0)),
            pl.BlockSpec((D, D), lambda i: (0, 0)),
        ],
        out_shape=[
            jax.ShapeDtypeStruct((D, D), jnp.float32),
            jax.ShapeDtypeStruct((D, D), jnp.float32),
        ],
    )(dsta.reshape(E, 1), dstb.reshape(E, 1))


_RB = 400


def _lin_body(a0, a1, cn, x, wl, wr, b, o):
    cnt = jnp.maximum(cn[...], 1.0)
    mean = (a0[0] + a1[0]) / cnt
    h = (jnp.dot(mean, wl[...], preferred_element_type=jnp.float32)
         + jnp.dot(x[...], wr[...], preferred_element_type=jnp.float32)
         + b[...])
    o[...] = jnp.maximum(h, 0.0)


def _sage_linear(agg, cnt, x, wl, wr, b):
    return pl.pallas_call(
        _lin_body,
        grid=(N // _RB,),
        in_specs=[
            pl.BlockSpec((1, _RB, D), lambda i: (0, i, 0)),
            pl.BlockSpec((1, _RB, D), lambda i: (1, i, 0)),
            pl.BlockSpec((_RB, 1), lambda i: (i, 0)),
            pl.BlockSpec((_RB, D), lambda i: (i, 0)),
            pl.BlockSpec((D, D), lambda i: (0, 0)),
            pl.BlockSpec((D, D), lambda i: (0, 0)),
            pl.BlockSpec((1, D), lambda i: (0, 0)),
        ],
        out_specs=pl.BlockSpec((_RB, D), lambda i: (i, 0)),
        out_shape=jax.ShapeDtypeStruct((N, D), jnp.float32),
    )(agg, agg, cnt, x, wl, wr, b.reshape(1, D))


def _final_body(a, cn, ht, wl, wr, b, gnw, gnb, gms, pw1, pb1, pw2, pb2, o):
    cnt = jnp.maximum(cn[...], 1.0)
    mean = (a[0] + a[1]) / cnt
    x = (jnp.dot(mean, wl[...], preferred_element_type=jnp.float32)
         + jnp.dot(ht[...], wr[...], preferred_element_type=jnp.float32)
         + b[...])
    mu = jnp.mean(x, axis=0, keepdims=True)
    cen = x - mu * gms[...]
    var = jnp.mean(cen * cen, axis=0, keepdims=True)
    x = gnw[...] * cen / jnp.sqrt(var + 1e-5) + gnb[...]
    x = jnp.maximum(jnp.dot(x, pw1[...], preferred_element_type=jnp.float32)
                    + pb1[...], 0.0)
    x = jnp.dot(x, pw2[...], preferred_element_type=jnp.float32) + pb2[...]
    nrm = jnp.sqrt(jnp.sum(x * x, axis=1, keepdims=True))
    o[...] = x / jnp.maximum(nrm, 1e-12)


def _final_stage(agg2, cntb, h_tab, wl, wr, b,
                 gnw, gnb, gms, pw1, pb1, pw2, pb2):
    r1 = lambda v: v.reshape(1, D)
    return pl.pallas_call(
        _final_body,
        grid=(1,),
        in_specs=[
            pl.BlockSpec((NC, N, D), lambda i: (0, 0, 0)),
            pl.BlockSpec((N, 1), lambda i: (0, 0)),
            pl.BlockSpec((N, D), lambda i: (0, 0)),
            pl.BlockSpec((D, D), lambda i: (0, 0)),
            pl.BlockSpec((D, D), lambda i: (0, 0)),
            pl.BlockSpec((1, D), lambda i: (0, 0)),
            pl.BlockSpec((1, D), lambda i: (0, 0)),
            pl.BlockSpec((1, D), lambda i: (0, 0)),
            pl.BlockSpec((1, D), lambda i: (0, 0)),
            pl.BlockSpec((D, D), lambda i: (0, 0)),
            pl.BlockSpec((1, D), lambda i: (0, 0)),
            pl.BlockSpec((D, D), lambda i: (0, 0)),
            pl.BlockSpec((1, D), lambda i: (0, 0)),
        ],
        out_specs=pl.BlockSpec((N, D), lambda i: (0, 0)),
        out_shape=jax.ShapeDtypeStruct((N, D), jnp.float32),
    )(agg2, cntb, h_tab, wl, wr, r1(b), r1(gnw), r1(gnb), r1(gms),
      pw1, r1(pb1), pw2, r1(pb2))


def kernel(x_table, x_column, edge_index_t2c, edge_index_c2t,
           Wl1_t2c, Wr1_t2c, b1_t2c, Wl1_c2t, Wr1_c2t, b1_c2t,
           Wl2_t2c, Wr2_t2c, b2_t2c, Wl2_c2t, Wr2_c2t, b2_c2t,
           gn_weight, gn_bias, gn_mean_scale, pW1, pb1, pW2, pb2):
    srcA, dstA = _pad_edges(edge_index_t2c)
    srcB, dstB = _pad_edges(edge_index_c2t)

    cntA_m, cntB_m = _cnt_tc(edge_index_t2c[1], edge_index_c2t[1])
    cntA = cntA_m.reshape(D * D, 1)
    cntB = cntB_m.reshape(D * D, 1)
    (aggA,) = _conv(x_table, srcA, dstA)
    (aggB,) = _conv(x_column, srcB, dstB)

    h_col = _sage_linear(aggA, cntA, x_column, Wl1_t2c, Wr1_t2c, b1_t2c)
    h_tab = _sage_linear(aggB, cntB, x_table, Wl1_c2t, Wr1_c2t, b1_c2t)

    (agg2,) = _conv(h_col, srcB, dstB)

    return _final_stage(agg2, cntB, h_tab, Wl2_c2t, Wr2_c2t, b2_c2t,
                        gn_weight, gn_bias, gn_mean_scale,
                        pW1, pb1, pW2, pb2)
